# baseline (speedup 1.0000x reference)
import functools

import jax
import jax.numpy as jnp
from jax import lax
from jax.experimental import pallas as pl
from jax.experimental.pallas import tpu as pltpu
from jax.experimental.pallas import tpu_sc as plsc

E, T, H, F, K = 8, 4096, 1024, 4096, 2
A = T * K
B = 512
P = A + E * B
NB = P // B
NBPAD = 32

NW = 32
CHUNK = A // NW
RCH = 64
RCHD = 16
NSLOT = 4
TOKW = T // NW

_INV_SQRT2 = 0.7071067811865476

_mesh = plsc.VectorSubcoreMesh(core_axis_name="c", subcore_axis_name="s")


def _wid():
    return lax.axis_index("s") * 2 + lax.axis_index("c")


def _lanes():
    return lax.broadcasted_iota(jnp.int32, (16,), 0)


def _vi(x):
    return jnp.full((16,), x, jnp.int32)


def _vf(x):
    return jnp.full((16,), x, jnp.float32)


_GDN = lax.GatherDimensionNumbers(
    offset_dims=(), collapsed_slice_dims=(0,), start_index_map=(0,))


def _lgather(x, idx):
    return lax.gather(x, idx[:, None], _GDN, (1,),
                      mode=lax.GatherScatterMode.PROMISE_IN_BOUNDS)


def _lanesum(x):
    lanes = _lanes()
    for d in (1, 2, 4, 8):
        x = x + _lgather(x, lanes ^ _vi(d))
    return x


def _prefix_incl(x):
    lanes = _lanes()
    zero = jnp.zeros((16,), x.dtype)
    for d in (1, 2, 4, 8):
        shifted = _lgather(x, jnp.maximum(lanes - _vi(d), _vi(0)))
        x = x + jnp.where(lanes >= _vi(d), shifted, zero)
    return x


@functools.partial(
    pl.kernel, mesh=_mesh,
    out_type=jax.ShapeDtypeStruct((NW, 16), jnp.int32),
    scratch_types=[pltpu.VMEM((CHUNK,), jnp.int32),
                   pltpu.VMEM((16,), jnp.int32)])
def _sc_hist(eid_hbm, hist_hbm, eid_v, cnt_v):
    wid = _wid()
    pltpu.sync_copy(eid_hbm.at[pl.ds(wid * CHUNK, CHUNK)], eid_v)
    lanes = _lanes()

    def body(j, cnt):
        v = eid_v[pl.ds(j * 16, 16)]
        for e in range(E):
            pc = _lanesum(jnp.where(v == _vi(e), _vi(1), _vi(0)))
            cnt = cnt + jnp.where(lanes == _vi(e), pc, _vi(0))
        return cnt

    cnt = lax.fori_loop(0, CHUNK // 16, body, jnp.zeros((16,), jnp.int32))
    cnt_v[...] = cnt
    pltpu.sync_copy(cnt_v, hist_hbm.at[wid])


@functools.partial(
    pl.kernel, mesh=_mesh,
    out_type=(jax.ShapeDtypeStruct((A,), jnp.int32),
              jax.ShapeDtypeStruct((NBPAD,), jnp.int32),
              jax.ShapeDtypeStruct((NBPAD,), jnp.int32),
              jax.ShapeDtypeStruct((P, H // 2), jnp.int32)),
    scratch_types=[pltpu.VMEM((CHUNK,), jnp.int32),
                   pltpu.VMEM((NW * 16,), jnp.int32),
                   pltpu.VMEM((CHUNK,), jnp.int32),
                   pltpu.VMEM((CHUNK,), jnp.int32),
                   pltpu.VMEM((CHUNK // RCHD, RCHD), jnp.int32),
                   pltpu.VMEM((NSLOT * RCHD, H // 2), jnp.int32),
                   pltpu.VMEM((NBPAD,), jnp.int32),
                   pltpu.VMEM((NBPAD,), jnp.int32),
                   pltpu.SemaphoreType.DMA,
                   pltpu.SemaphoreType.DMA,
                   pltpu.SemaphoreType.DMA,
                   pltpu.SemaphoreType.DMA])
def _sc_dispatch(eid_hbm, hist_hbm, hidden_hbm, pos_hbm, be_hbm, valid_hbm,
                 xs_hbm, eid_v, hist_v, pos_v, tok_v, pidx_v, rows_all, be_v,
                 valid_v, sem0, sem1, sem2, sem3):
    wid = _wid()
    lanes = _lanes()
    pltpu.sync_copy(eid_hbm.at[pl.ds(wid * CHUNK, CHUNK)], eid_v)
    pltpu.sync_copy(hist_hbm, hist_v)

    def hbody(w, carry):
        tot, mine = carry
        row = hist_v[pl.ds(w * 16, 16)]
        sel = jnp.where(w < wid, 1, 0)
        return tot + row, mine + row * _vi(sel)

    zeros16 = jnp.zeros((16,), jnp.int32)
    tot, mine = lax.fori_loop(0, NW, hbody, (zeros16, zeros16))

    padded = ((tot + _vi(B - 1)) >> 9) << 9
    ends = _prefix_incl(padded)
    bases = ends - padded
    cur0 = bases + mine

    def pbody(j, cur):
        v = eid_v[pl.ds(j * 16, 16)]
        pos = _vi(0)
        for e in range(E):
            m = v == _vi(e)
            m01 = jnp.where(m, _vi(1), _vi(0))
            r = _prefix_incl(m01)
            base_e = _lanesum(jnp.where(lanes == _vi(e), cur, _vi(0)))
            pos = jnp.where(m, base_e - _vi(1) + r, pos)
            pc = _lgather(r, _vi(15))
            cur = cur + jnp.where(lanes == _vi(e), pc, _vi(0))
        pos_v[pl.ds(j * 16, 16)] = pos
        return cur

    lax.fori_loop(0, CHUNK // 16, pbody, cur0)
    pltpu.sync_copy(pos_v, pos_hbm.at[pl.ds(wid * CHUNK, CHUNK)])

    def tbody(j, _):
        base = wid * CHUNK + j * 16
        tok_v[pl.ds(j * 16, 16)] = (_vi(base) + lanes) >> _vi(1)
        return 0

    lax.fori_loop(0, CHUNK // 16, tbody, 0)

    for cb in range(CHUNK // RCHD):
        for g in range(RCHD // 16):
            pidx_v[cb, pl.ds(g * 16, 16)] = pos_v[pl.ds(cb * RCHD + g * 16,
                                                        16)]

    nch = CHUNK // RCHD
    sems = (sem0, sem1, sem2, sem3)

    def _slot(s):
        return rows_all.at[pl.ds(s * RCHD, RCHD)]

    def _gather(cb, s):
        return pltpu.async_copy(
            hidden_hbm.at[tok_v.at[pl.ds(cb * RCHD, RCHD)]],
            _slot(s), sems[s])

    gpend = {0: _gather(0, 0), 1: _gather(1, 1)}
    spend = [None] * NSLOT
    for cb in range(nch):
        s = cb % NSLOT
        gpend.pop(cb).wait()
        spend[s] = pltpu.async_copy(_slot(s), xs_hbm.at[pidx_v.at[cb]],
                                    sems[s])
        la = cb + 2
        if la < nch:
            sl = la % NSLOT
            if spend[sl] is not None:
                spend[sl].wait()
                spend[sl] = None
            gpend[la] = _gather(la, sl)
    for s in range(NSLOT):
        if spend[s] is not None:
            spend[s].wait()

    rends = bases + tot
    for nbc in range(NBPAD // 16):
        blk = (_vi(nbc * 16) + lanes) * _vi(B)
        cntv = _vi(0)
        for e in range(E):
            end_e = _lanesum(jnp.where(lanes == _vi(e), ends, _vi(0)))
            cntv = cntv + jnp.where(blk >= end_e, _vi(1), _vi(0))
        be16 = jnp.minimum(cntv, _vi(E - 1))
        be_v[pl.ds(nbc * 16, 16)] = be16
        rend_g = _lgather(rends, be16)
        valid_v[pl.ds(nbc * 16, 16)] = jnp.where(blk < rend_g, _vi(1),
                                                 _vi(0))

    @pl.when(wid == 0)
    def _publish():
        pltpu.sync_copy(be_v, be_hbm)
        pltpu.sync_copy(valid_v, valid_hbm)


@functools.partial(
    pl.kernel, mesh=_mesh,
    out_type=jax.ShapeDtypeStruct((T, H), jnp.float32),
    scratch_types=[pltpu.VMEM((CHUNK,), jnp.float32),
                   pltpu.VMEM((CHUNK,), jnp.int32),
                   pltpu.VMEM((RCH, H), jnp.float32),
                   pltpu.VMEM((RCH // 2, H), jnp.float32),
                   pltpu.SemaphoreType.DMA])
def _sc_combine(y_hbm, pos_hbm, p_hbm, out_hbm, p_v, pos_v, rows_v, out_v,
                sem):
    wid = _wid()
    lanes = _lanes()
    abase = wid * CHUNK
    tbase = wid * TOKW
    pltpu.sync_copy(pos_hbm.at[pl.ds(abase, CHUNK)], pos_v)
    pltpu.sync_copy(p_hbm.at[pl.ds(abase, CHUNK)], p_v)
    for cb in range(CHUNK // RCH):
        pltpu.async_copy(
            y_hbm.at[pos_v.at[pl.ds(cb * RCH, RCH)]], rows_v, sem).wait()
        for sg in range(RCH // 16):
            pv = p_v[pl.ds(cb * RCH + sg * 16, 16)]
            p0 = [_lgather(pv, _vi(2 * t)) for t in range(8)]
            p1 = [_lgather(pv, _vi(2 * t + 1)) for t in range(8)]

            def qbody(q, _, sg=sg, p0=p0, p1=p1):
                sl = pl.ds(q * 16, 16)
                for t in range(8):
                    r0 = rows_v[sg * 16 + 2 * t, sl]
                    r1 = rows_v[sg * 16 + 2 * t + 1, sl]
                    out_v[sg * 8 + t, sl] = p0[t] * r0 + p1[t] * r1
                return 0

            lax.fori_loop(0, H // 16, qbody, 0)
        pltpu.sync_copy(out_v, out_hbm.at[pl.ds(tbase + cb * (RCH // 2),
                                                RCH // 2)])


def _gelu_exact(x):
    return 0.5 * x * (1.0 + lax.erf(x * _INV_SQRT2))


def _fc1_body(be_ref, valid_ref, x_ref, w1_ref, b1_ref, h_ref,
              wbf_ref, last_ref):
    i = pl.program_id(0)

    @pl.when(valid_ref[i] != 0)
    def _compute():
        @pl.when(i == 0)
        def _init():
            last_ref[0] = -1

        @pl.when(be_ref[i] != last_ref[0])
        def _recast():
            wbf_ref[...] = w1_ref[0].astype(jnp.bfloat16)
            last_ref[0] = be_ref[i]

        h = (jnp.dot(x_ref[...], wbf_ref[...],
                     preferred_element_type=jnp.float32)
             + b1_ref[0])
        h_ref[...] = _gelu_exact(h).astype(jnp.bfloat16)


def _fc2_body(be_ref, valid_ref, h_ref, w2_ref, b2_ref, y_ref,
              wbf_ref, last_ref):
    i = pl.program_id(0)

    @pl.when(valid_ref[i] != 0)
    def _compute():
        @pl.when(i == 0)
        def _init():
            last_ref[0] = -1

        @pl.when(be_ref[i] != last_ref[0])
        def _recast():
            wbf_ref[...] = w2_ref[0].astype(jnp.bfloat16)
            last_ref[0] = be_ref[i]

        y_ref[...] = (jnp.dot(h_ref[...], wbf_ref[...],
                              preferred_element_type=jnp.float32)
                      + b2_ref[0])


def _grouped_mlp(x_sorted, block_expert, block_valid, W1, b1r, W2, b2r):
    fc1_spec = pltpu.PrefetchScalarGridSpec(
        num_scalar_prefetch=2,
        grid=(NB,),
        in_specs=[
            pl.BlockSpec((B, H), lambda i, be, va: (i, 0)),
            pl.BlockSpec((1, H, F), lambda i, be, va: (be[i], 0, 0)),
            pl.BlockSpec((1, 1, F), lambda i, be, va: (be[i], 0, 0)),
        ],
        out_specs=pl.BlockSpec((B, F), lambda i, be, va: (i, 0)),
        scratch_shapes=[pltpu.VMEM((H, F), jnp.bfloat16),
                        pltpu.SMEM((1,), jnp.int32)],
    )
    h_all = pl.pallas_call(
        _fc1_body,
        grid_spec=fc1_spec,
        out_shape=jax.ShapeDtypeStruct((P, F), jnp.bfloat16),
        compiler_params=pltpu.CompilerParams(
            dimension_semantics=("arbitrary",),
        ),
    )(block_expert, block_valid, x_sorted, W1, b1r)
    fc2_spec = pltpu.PrefetchScalarGridSpec(
        num_scalar_prefetch=2,
        grid=(NB,),
        in_specs=[
            pl.BlockSpec((B, F), lambda i, be, va: (i, 0)),
            pl.BlockSpec((1, F, H), lambda i, be, va: (be[i], 0, 0)),
            pl.BlockSpec((1, 1, H), lambda i, be, va: (be[i], 0, 0)),
        ],
        out_specs=pl.BlockSpec((B, H), lambda i, be, va: (i, 0)),
        scratch_shapes=[pltpu.VMEM((F, H), jnp.bfloat16),
                        pltpu.SMEM((1,), jnp.int32)],
    )
    return pl.pallas_call(
        _fc2_body,
        grid_spec=fc2_spec,
        out_shape=jax.ShapeDtypeStruct((P, H), jnp.float32),
        compiler_params=pltpu.CompilerParams(
            dimension_semantics=("arbitrary",),
        ),
    )(block_expert, block_valid, h_all, W2, b2r)


@jax.jit
def _moe(hidden_flat, probs, indices, W1, b1, W2, b2):
    eid = indices.astype(jnp.int32).reshape(A)
    p_flat = probs.reshape(A)
    b1r = b1.reshape(E, 1, F)
    b2r = b2.reshape(E, 1, H)
    hidden_bf = hidden_flat.astype(jnp.bfloat16)
    hid32 = lax.bitcast_convert_type(
        hidden_bf.reshape(T, H // 2, 2), jnp.int32)
    hist = _sc_hist(eid)
    pos, be, valid, xs32 = _sc_dispatch(eid, hist.reshape(NW * 16), hid32)
    x_sorted = lax.bitcast_convert_type(xs32, jnp.bfloat16).reshape(P, H)
    y_all = _grouped_mlp(x_sorted, be, valid, W1, b1r, W2, b2r)
    return _sc_combine(y_all, pos, p_flat)


def kernel(hidden_flat, probs, indices, W1, b1, W2, b2):
    return _moe(hidden_flat, probs, indices, W1, b1, W2, b2)


# speedup vs baseline: 1.8608x; 1.8608x over previous
import functools

import jax
import jax.numpy as jnp
from jax import lax
from jax.experimental import pallas as pl
from jax.experimental.pallas import tpu as pltpu
from jax.experimental.pallas import tpu_sc as plsc

E, T, H, F, K = 8, 4096, 1024, 4096, 2
A = T * K
B = 512
P = A + E * B
NB = P // B
NBPAD = 32

NW = 32
CHUNK = A // NW
RCH = 64
RCHD = 16
NSLOT = 4
TOKW = T // NW

_INV_SQRT2 = 0.7071067811865476

_mesh = plsc.VectorSubcoreMesh(core_axis_name="c", subcore_axis_name="s")


def _wid():
    return lax.axis_index("s") * 2 + lax.axis_index("c")


def _lanes():
    return lax.broadcasted_iota(jnp.int32, (16,), 0)


def _vi(x):
    return jnp.full((16,), x, jnp.int32)


def _vf(x):
    return jnp.full((16,), x, jnp.float32)


_GDN = lax.GatherDimensionNumbers(
    offset_dims=(), collapsed_slice_dims=(0,), start_index_map=(0,))


def _lgather(x, idx):
    return lax.gather(x, idx[:, None], _GDN, (1,),
                      mode=lax.GatherScatterMode.PROMISE_IN_BOUNDS)


def _lanesum(x):
    lanes = _lanes()
    for d in (1, 2, 4, 8):
        x = x + _lgather(x, lanes ^ _vi(d))
    return x


def _prefix_incl(x):
    lanes = _lanes()
    zero = jnp.zeros((16,), x.dtype)
    for d in (1, 2, 4, 8):
        shifted = _lgather(x, jnp.maximum(lanes - _vi(d), _vi(0)))
        x = x + jnp.where(lanes >= _vi(d), shifted, zero)
    return x


@functools.partial(
    pl.kernel, mesh=_mesh,
    out_type=jax.ShapeDtypeStruct((NW, 16), jnp.int32),
    scratch_types=[pltpu.VMEM((CHUNK,), jnp.int32),
                   pltpu.VMEM((16,), jnp.int32)])
def _sc_hist(eid_hbm, hist_hbm, eid_v, cnt_v):
    wid = _wid()
    pltpu.sync_copy(eid_hbm.at[pl.ds(wid * CHUNK, CHUNK)], eid_v)
    lanes = _lanes()

    def body(j, cnt):
        v = eid_v[pl.ds(j * 16, 16)]
        for e in range(E):
            pc = _lanesum(jnp.where(v == _vi(e), _vi(1), _vi(0)))
            cnt = cnt + jnp.where(lanes == _vi(e), pc, _vi(0))
        return cnt

    cnt = lax.fori_loop(0, CHUNK // 16, body, jnp.zeros((16,), jnp.int32))
    cnt_v[...] = cnt
    pltpu.sync_copy(cnt_v, hist_hbm.at[wid])


@functools.partial(
    pl.kernel, mesh=_mesh,
    out_type=(jax.ShapeDtypeStruct((A,), jnp.int32),
              jax.ShapeDtypeStruct((NBPAD,), jnp.int32),
              jax.ShapeDtypeStruct((NBPAD,), jnp.int32),
              jax.ShapeDtypeStruct((P, H), jnp.float32)),
    scratch_types=[pltpu.VMEM((CHUNK,), jnp.int32),
                   pltpu.VMEM((NW * 16,), jnp.int32),
                   pltpu.VMEM((CHUNK,), jnp.int32),
                   pltpu.VMEM((CHUNK,), jnp.int32),
                   pltpu.VMEM((CHUNK // RCHD, RCHD), jnp.int32),
                   pltpu.VMEM((NSLOT * RCHD, H), jnp.float32),
                   pltpu.VMEM((NBPAD,), jnp.int32),
                   pltpu.VMEM((NBPAD,), jnp.int32),
                   pltpu.SemaphoreType.DMA,
                   pltpu.SemaphoreType.DMA,
                   pltpu.SemaphoreType.DMA,
                   pltpu.SemaphoreType.DMA])
def _sc_dispatch(eid_hbm, hist_hbm, hidden_hbm, pos_hbm, be_hbm, valid_hbm,
                 xs_hbm, eid_v, hist_v, pos_v, tok_v, pidx_v, rows_all, be_v,
                 valid_v, sem0, sem1, sem2, sem3):
    wid = _wid()
    lanes = _lanes()
    pltpu.sync_copy(eid_hbm.at[pl.ds(wid * CHUNK, CHUNK)], eid_v)
    pltpu.sync_copy(hist_hbm, hist_v)

    def hbody(w, carry):
        tot, mine = carry
        row = hist_v[pl.ds(w * 16, 16)]
        sel = jnp.where(w < wid, 1, 0)
        return tot + row, mine + row * _vi(sel)

    zeros16 = jnp.zeros((16,), jnp.int32)
    tot, mine = lax.fori_loop(0, NW, hbody, (zeros16, zeros16))

    padded = ((tot + _vi(B - 1)) >> 9) << 9
    ends = _prefix_incl(padded)
    bases = ends - padded
    cur0 = bases + mine

    def pbody(j, cur):
        v = eid_v[pl.ds(j * 16, 16)]
        pos = _vi(0)
        for e in range(E):
            m = v == _vi(e)
            m01 = jnp.where(m, _vi(1), _vi(0))
            r = _prefix_incl(m01)
            base_e = _lanesum(jnp.where(lanes == _vi(e), cur, _vi(0)))
            pos = jnp.where(m, base_e - _vi(1) + r, pos)
            pc = _lgather(r, _vi(15))
            cur = cur + jnp.where(lanes == _vi(e), pc, _vi(0))
        pos_v[pl.ds(j * 16, 16)] = pos
        return cur

    lax.fori_loop(0, CHUNK // 16, pbody, cur0)
    pltpu.sync_copy(pos_v, pos_hbm.at[pl.ds(wid * CHUNK, CHUNK)])

    def tbody(j, _):
        base = wid * CHUNK + j * 16
        tok_v[pl.ds(j * 16, 16)] = (_vi(base) + lanes) >> _vi(1)
        return 0

    lax.fori_loop(0, CHUNK // 16, tbody, 0)

    for cb in range(CHUNK // RCHD):
        for g in range(RCHD // 16):
            pidx_v[cb, pl.ds(g * 16, 16)] = pos_v[pl.ds(cb * RCHD + g * 16,
                                                        16)]

    nch = CHUNK // RCHD
    sems = (sem0, sem1, sem2, sem3)

    def _slot(s):
        return rows_all.at[pl.ds(s * RCHD, RCHD)]

    def _gather(cb, s):
        return pltpu.async_copy(
            hidden_hbm.at[tok_v.at[pl.ds(cb * RCHD, RCHD)]],
            _slot(s), sems[s])

    gpend = {0: _gather(0, 0), 1: _gather(1, 1)}
    spend = [None] * NSLOT
    for cb in range(nch):
        s = cb % NSLOT
        gpend.pop(cb).wait()
        spend[s] = pltpu.async_copy(_slot(s), xs_hbm.at[pidx_v.at[cb]],
                                    sems[s])
        la = cb + 2
        if la < nch:
            sl = la % NSLOT
            if spend[sl] is not None:
                spend[sl].wait()
                spend[sl] = None
            gpend[la] = _gather(la, sl)
    for s in range(NSLOT):
        if spend[s] is not None:
            spend[s].wait()

    rends = bases + tot
    for nbc in range(NBPAD // 16):
        blk = (_vi(nbc * 16) + lanes) * _vi(B)
        cntv = _vi(0)
        for e in range(E):
            end_e = _lanesum(jnp.where(lanes == _vi(e), ends, _vi(0)))
            cntv = cntv + jnp.where(blk >= end_e, _vi(1), _vi(0))
        be16 = jnp.minimum(cntv, _vi(E - 1))
        be_v[pl.ds(nbc * 16, 16)] = be16
        rend_g = _lgather(rends, be16)
        valid_v[pl.ds(nbc * 16, 16)] = jnp.where(blk < rend_g, _vi(1),
                                                 _vi(0))

    @pl.when(wid == 0)
    def _publish():
        pltpu.sync_copy(be_v, be_hbm)
        pltpu.sync_copy(valid_v, valid_hbm)


@functools.partial(
    pl.kernel, mesh=_mesh,
    out_type=jax.ShapeDtypeStruct((T, H), jnp.float32),
    scratch_types=[pltpu.VMEM((CHUNK,), jnp.float32),
                   pltpu.VMEM((CHUNK,), jnp.int32),
                   pltpu.VMEM((RCH, H), jnp.float32),
                   pltpu.VMEM((RCH // 2, H), jnp.float32),
                   pltpu.SemaphoreType.DMA])
def _sc_combine(y_hbm, pos_hbm, p_hbm, out_hbm, p_v, pos_v, rows_v, out_v,
                sem):
    wid = _wid()
    lanes = _lanes()
    abase = wid * CHUNK
    tbase = wid * TOKW
    pltpu.sync_copy(pos_hbm.at[pl.ds(abase, CHUNK)], pos_v)
    pltpu.sync_copy(p_hbm.at[pl.ds(abase, CHUNK)], p_v)
    for cb in range(CHUNK // RCH):
        pltpu.async_copy(
            y_hbm.at[pos_v.at[pl.ds(cb * RCH, RCH)]], rows_v, sem).wait()
        for sg in range(RCH // 16):
            pv = p_v[pl.ds(cb * RCH + sg * 16, 16)]
            p0 = [_lgather(pv, _vi(2 * t)) for t in range(8)]
            p1 = [_lgather(pv, _vi(2 * t + 1)) for t in range(8)]

            def qbody(q, _, sg=sg, p0=p0, p1=p1):
                sl = pl.ds(q * 16, 16)
                for t in range(8):
                    r0 = rows_v[sg * 16 + 2 * t, sl]
                    r1 = rows_v[sg * 16 + 2 * t + 1, sl]
                    out_v[sg * 8 + t, sl] = p0[t] * r0 + p1[t] * r1
                return 0

            lax.fori_loop(0, H // 16, qbody, 0)
        pltpu.sync_copy(out_v, out_hbm.at[pl.ds(tbase + cb * (RCH // 2),
                                                RCH // 2)])


def _gelu_exact(x):
    return 0.5 * x * (1.0 + lax.erf(x * _INV_SQRT2))


def _fc1_body(be_ref, valid_ref, x_ref, w1_ref, b1_ref, h_ref,
              wbf_ref, last_ref):
    i = pl.program_id(0)

    @pl.when(valid_ref[i] != 0)
    def _compute():
        @pl.when(i == 0)
        def _init():
            last_ref[0] = -1

        @pl.when(be_ref[i] != last_ref[0])
        def _recast():
            wbf_ref[...] = w1_ref[0].astype(jnp.bfloat16)
            last_ref[0] = be_ref[i]

        h = (jnp.dot(x_ref[...].astype(jnp.bfloat16), wbf_ref[...],
                     preferred_element_type=jnp.float32)
             + b1_ref[0])
        h_ref[...] = _gelu_exact(h).astype(jnp.bfloat16)


def _fc2_body(be_ref, valid_ref, h_ref, w2_ref, b2_ref, y_ref,
              wbf_ref, last_ref):
    i = pl.program_id(0)

    @pl.when(valid_ref[i] != 0)
    def _compute():
        @pl.when(i == 0)
        def _init():
            last_ref[0] = -1

        @pl.when(be_ref[i] != last_ref[0])
        def _recast():
            wbf_ref[...] = w2_ref[0].astype(jnp.bfloat16)
            last_ref[0] = be_ref[i]

        y_ref[...] = (jnp.dot(h_ref[...], wbf_ref[...],
                              preferred_element_type=jnp.float32)
                      + b2_ref[0])


def _grouped_mlp(x_sorted, block_expert, block_valid, W1, b1r, W2, b2r):
    fc1_spec = pltpu.PrefetchScalarGridSpec(
        num_scalar_prefetch=2,
        grid=(NB,),
        in_specs=[
            pl.BlockSpec((B, H), lambda i, be, va: (i, 0)),
            pl.BlockSpec((1, H, F), lambda i, be, va: (be[i], 0, 0)),
            pl.BlockSpec((1, 1, F), lambda i, be, va: (be[i], 0, 0)),
        ],
        out_specs=pl.BlockSpec((B, F), lambda i, be, va: (i, 0)),
        scratch_shapes=[pltpu.VMEM((H, F), jnp.bfloat16),
                        pltpu.SMEM((1,), jnp.int32)],
    )
    h_all = pl.pallas_call(
        _fc1_body,
        grid_spec=fc1_spec,
        out_shape=jax.ShapeDtypeStruct((P, F), jnp.bfloat16),
        compiler_params=pltpu.CompilerParams(
            dimension_semantics=("arbitrary",),
        ),
    )(block_expert, block_valid, x_sorted, W1, b1r)
    fc2_spec = pltpu.PrefetchScalarGridSpec(
        num_scalar_prefetch=2,
        grid=(NB,),
        in_specs=[
            pl.BlockSpec((B, F), lambda i, be, va: (i, 0)),
            pl.BlockSpec((1, F, H), lambda i, be, va: (be[i], 0, 0)),
            pl.BlockSpec((1, 1, H), lambda i, be, va: (be[i], 0, 0)),
        ],
        out_specs=pl.BlockSpec((B, H), lambda i, be, va: (i, 0)),
        scratch_shapes=[pltpu.VMEM((F, H), jnp.bfloat16),
                        pltpu.SMEM((1,), jnp.int32)],
    )
    return pl.pallas_call(
        _fc2_body,
        grid_spec=fc2_spec,
        out_shape=jax.ShapeDtypeStruct((P, H), jnp.float32),
        compiler_params=pltpu.CompilerParams(
            dimension_semantics=("arbitrary",),
        ),
    )(block_expert, block_valid, h_all, W2, b2r)


@jax.jit
def _moe(hidden_flat, probs, indices, W1, b1, W2, b2):
    eid = indices.astype(jnp.int32).reshape(A)
    p_flat = probs.reshape(A)
    b1r = b1.reshape(E, 1, F)
    b2r = b2.reshape(E, 1, H)
    hist = _sc_hist(eid)
    pos, be, valid, x_sorted = _sc_dispatch(eid, hist.reshape(NW * 16),
                                            hidden_flat)
    y_all = _grouped_mlp(x_sorted, be, valid, W1, b1r, W2, b2r)
    return _sc_combine(y_all, pos, p_flat)


def kernel(hidden_flat, probs, indices, W1, b1, W2, b2):
    return _moe(hidden_flat, probs, indices, W1, b1, W2, b2)


# speedup vs baseline: 1.9252x; 1.0346x over previous
import functools

import jax
import jax.numpy as jnp
from jax import lax
from jax.experimental import pallas as pl
from jax.experimental.pallas import tpu as pltpu
from jax.experimental.pallas import tpu_sc as plsc

E, T, H, F, K = 8, 4096, 1024, 4096, 2
A = T * K
B = 512
P = A + E * B
NB = P // B
NBPAD = 32

NW = 32
CHUNK = A // NW
RCH = 64
RCHD = 16
NSLOT = 4
TOKW = T // NW

_INV_SQRT2 = 0.7071067811865476

_mesh = plsc.VectorSubcoreMesh(core_axis_name="c", subcore_axis_name="s")


def _wid():
    return lax.axis_index("s") * 2 + lax.axis_index("c")


def _lanes():
    return lax.broadcasted_iota(jnp.int32, (16,), 0)


def _vi(x):
    return jnp.full((16,), x, jnp.int32)


def _vf(x):
    return jnp.full((16,), x, jnp.float32)


_GDN = lax.GatherDimensionNumbers(
    offset_dims=(), collapsed_slice_dims=(0,), start_index_map=(0,))


def _lgather(x, idx):
    return lax.gather(x, idx[:, None], _GDN, (1,),
                      mode=lax.GatherScatterMode.PROMISE_IN_BOUNDS)


def _lanesum(x):
    lanes = _lanes()
    for d in (1, 2, 4, 8):
        x = x + _lgather(x, lanes ^ _vi(d))
    return x


def _prefix_incl(x):
    lanes = _lanes()
    zero = jnp.zeros((16,), x.dtype)
    for d in (1, 2, 4, 8):
        shifted = _lgather(x, jnp.maximum(lanes - _vi(d), _vi(0)))
        x = x + jnp.where(lanes >= _vi(d), shifted, zero)
    return x


@functools.partial(
    pl.kernel, mesh=_mesh,
    out_type=jax.ShapeDtypeStruct((NW, 16), jnp.int32),
    scratch_types=[pltpu.VMEM((CHUNK,), jnp.int32),
                   pltpu.VMEM((16,), jnp.int32)])
def _sc_hist(eid_hbm, hist_hbm, eid_v, cnt_v):
    wid = _wid()
    pltpu.sync_copy(eid_hbm.at[pl.ds(wid * CHUNK, CHUNK)], eid_v)
    lanes = _lanes()

    def body(j, cnt):
        v = eid_v[pl.ds(j * 16, 16)]
        for e in range(E):
            pc = _lanesum(jnp.where(v == _vi(e), _vi(1), _vi(0)))
            cnt = cnt + jnp.where(lanes == _vi(e), pc, _vi(0))
        return cnt

    cnt = lax.fori_loop(0, CHUNK // 16, body, jnp.zeros((16,), jnp.int32))
    cnt_v[...] = cnt
    pltpu.sync_copy(cnt_v, hist_hbm.at[wid])


@functools.partial(
    pl.kernel, mesh=_mesh,
    out_type=(jax.ShapeDtypeStruct((A,), jnp.int32),
              jax.ShapeDtypeStruct((NBPAD,), jnp.int32),
              jax.ShapeDtypeStruct((NBPAD,), jnp.int32),
              jax.ShapeDtypeStruct((P, H), jnp.float32)),
    scratch_types=[pltpu.VMEM((CHUNK,), jnp.int32),
                   pltpu.VMEM((NW * 16,), jnp.int32),
                   pltpu.VMEM((CHUNK,), jnp.int32),
                   pltpu.VMEM((CHUNK,), jnp.int32),
                   pltpu.VMEM((CHUNK // RCHD, RCHD), jnp.int32),
                   pltpu.VMEM((NSLOT * RCHD, H), jnp.float32),
                   pltpu.VMEM((NBPAD,), jnp.int32),
                   pltpu.VMEM((NBPAD,), jnp.int32),
                   pltpu.SemaphoreType.DMA,
                   pltpu.SemaphoreType.DMA,
                   pltpu.SemaphoreType.DMA,
                   pltpu.SemaphoreType.DMA])
def _sc_dispatch(eid_hbm, hist_hbm, hidden_hbm, pos_hbm, be_hbm, valid_hbm,
                 xs_hbm, eid_v, hist_v, pos_v, tok_v, pidx_v, rows_all, be_v,
                 valid_v, sem0, sem1, sem2, sem3):
    wid = _wid()
    lanes = _lanes()
    pltpu.sync_copy(eid_hbm.at[pl.ds(wid * CHUNK, CHUNK)], eid_v)
    pltpu.sync_copy(hist_hbm, hist_v)

    def hbody(w, carry):
        tot, mine = carry
        row = hist_v[pl.ds(w * 16, 16)]
        sel = jnp.where(w < wid, 1, 0)
        return tot + row, mine + row * _vi(sel)

    zeros16 = jnp.zeros((16,), jnp.int32)
    tot, mine = lax.fori_loop(0, NW, hbody, (zeros16, zeros16))

    padded = ((tot + _vi(B - 1)) >> 9) << 9
    ends = _prefix_incl(padded)
    bases = ends - padded
    cur0 = bases + mine

    def pbody(j, cur):
        v = eid_v[pl.ds(j * 16, 16)]
        pos = _vi(0)
        for e in range(E):
            m = v == _vi(e)
            m01 = jnp.where(m, _vi(1), _vi(0))
            r = _prefix_incl(m01)
            base_e = _lanesum(jnp.where(lanes == _vi(e), cur, _vi(0)))
            pos = jnp.where(m, base_e - _vi(1) + r, pos)
            pc = _lgather(r, _vi(15))
            cur = cur + jnp.where(lanes == _vi(e), pc, _vi(0))
        pos_v[pl.ds(j * 16, 16)] = pos
        return cur

    lax.fori_loop(0, CHUNK // 16, pbody, cur0)
    pltpu.sync_copy(pos_v, pos_hbm.at[pl.ds(wid * CHUNK, CHUNK)])

    def tbody(j, _):
        base = wid * CHUNK + j * 16
        tok_v[pl.ds(j * 16, 16)] = (_vi(base) + lanes) >> _vi(1)
        return 0

    lax.fori_loop(0, CHUNK // 16, tbody, 0)

    for cb in range(CHUNK // RCHD):
        for g in range(RCHD // 16):
            pidx_v[cb, pl.ds(g * 16, 16)] = pos_v[pl.ds(cb * RCHD + g * 16,
                                                        16)]

    nch = CHUNK // RCHD
    sems = (sem0, sem1, sem2, sem3)

    def _slot(s):
        return rows_all.at[pl.ds(s * RCHD, RCHD)]

    def _gather(cb, s):
        return pltpu.async_copy(
            hidden_hbm.at[tok_v.at[pl.ds(cb * RCHD, RCHD)]],
            _slot(s), sems[s])

    gpend = {0: _gather(0, 0), 1: _gather(1, 1)}
    spend = [None] * NSLOT
    for cb in range(nch):
        s = cb % NSLOT
        gpend.pop(cb).wait()
        spend[s] = pltpu.async_copy(_slot(s), xs_hbm.at[pidx_v.at[cb]],
                                    sems[s])
        la = cb + 2
        if la < nch:
            sl = la % NSLOT
            if spend[sl] is not None:
                spend[sl].wait()
                spend[sl] = None
            gpend[la] = _gather(la, sl)
    for s in range(NSLOT):
        if spend[s] is not None:
            spend[s].wait()

    rends = bases + tot
    for nbc in range(NBPAD // 16):
        blk = (_vi(nbc * 16) + lanes) * _vi(B)
        cntv = _vi(0)
        for e in range(E):
            end_e = _lanesum(jnp.where(lanes == _vi(e), ends, _vi(0)))
            cntv = cntv + jnp.where(blk >= end_e, _vi(1), _vi(0))
        be16 = jnp.minimum(cntv, _vi(E - 1))
        be_v[pl.ds(nbc * 16, 16)] = be16
        rend_g = _lgather(rends, be16)
        valid_v[pl.ds(nbc * 16, 16)] = jnp.where(blk < rend_g, _vi(1),
                                                 _vi(0))

    @pl.when(wid == 0)
    def _publish():
        pltpu.sync_copy(be_v, be_hbm)
        pltpu.sync_copy(valid_v, valid_hbm)


@functools.partial(
    pl.kernel, mesh=_mesh,
    out_type=jax.ShapeDtypeStruct((T, H), jnp.float32),
    scratch_types=[pltpu.VMEM((CHUNK,), jnp.float32),
                   pltpu.VMEM((CHUNK,), jnp.int32),
                   pltpu.VMEM((RCH, H), jnp.float32),
                   pltpu.VMEM((RCH // 2, H), jnp.float32),
                   pltpu.SemaphoreType.DMA])
def _sc_combine(y_hbm, pos_hbm, p_hbm, out_hbm, p_v, pos_v, rows_v, out_v,
                sem):
    wid = _wid()
    lanes = _lanes()
    abase = wid * CHUNK
    tbase = wid * TOKW
    pltpu.sync_copy(pos_hbm.at[pl.ds(abase, CHUNK)], pos_v)
    pltpu.sync_copy(p_hbm.at[pl.ds(abase, CHUNK)], p_v)
    for cb in range(CHUNK // RCH):
        pltpu.async_copy(
            y_hbm.at[pos_v.at[pl.ds(cb * RCH, RCH)]], rows_v, sem).wait()
        for sg in range(RCH // 16):
            pv = p_v[pl.ds(cb * RCH + sg * 16, 16)]
            p0 = [_lgather(pv, _vi(2 * t)) for t in range(8)]
            p1 = [_lgather(pv, _vi(2 * t + 1)) for t in range(8)]

            def qbody(q, _, sg=sg, p0=p0, p1=p1):
                sl = pl.ds(q * 16, 16)
                for t in range(8):
                    r0 = rows_v[sg * 16 + 2 * t, sl]
                    r1 = rows_v[sg * 16 + 2 * t + 1, sl]
                    out_v[sg * 8 + t, sl] = p0[t] * r0 + p1[t] * r1
                return 0

            lax.fori_loop(0, H // 16, qbody, 0)
        pltpu.sync_copy(out_v, out_hbm.at[pl.ds(tbase + cb * (RCH // 2),
                                                RCH // 2)])


def _gelu_exact(x):
    return 0.5 * x * (1.0 + lax.erf(x * _INV_SQRT2))


def _fc1_body(be_ref, valid_ref, x_ref, w1_ref, b1_ref, h_ref):
    i = pl.program_id(0)

    @pl.when(valid_ref[i] != 0)
    def _compute():
        h = (jnp.dot(x_ref[...].astype(jnp.bfloat16),
                     w1_ref[0].astype(jnp.bfloat16),
                     preferred_element_type=jnp.float32)
             + b1_ref[0])
        h_ref[...] = _gelu_exact(h).astype(jnp.bfloat16)


def _fc2_body(be_ref, valid_ref, h_ref, w2_ref, b2_ref, y_ref):
    i = pl.program_id(0)

    @pl.when(valid_ref[i] != 0)
    def _compute():
        y_ref[...] = (jnp.dot(h_ref[...], w2_ref[0].astype(jnp.bfloat16),
                              preferred_element_type=jnp.float32)
                      + b2_ref[0])


def _grouped_mlp(x_sorted, block_expert, block_valid, W1, b1r, W2, b2r):
    fc1_spec = pltpu.PrefetchScalarGridSpec(
        num_scalar_prefetch=2,
        grid=(NB,),
        in_specs=[
            pl.BlockSpec((B, H), lambda i, be, va: (i, 0)),
            pl.BlockSpec((1, H, F), lambda i, be, va: (be[i], 0, 0)),
            pl.BlockSpec((1, 1, F), lambda i, be, va: (be[i], 0, 0)),
        ],
        out_specs=pl.BlockSpec((B, F), lambda i, be, va: (i, 0)),
    )
    h_all = pl.pallas_call(
        _fc1_body,
        grid_spec=fc1_spec,
        out_shape=jax.ShapeDtypeStruct((P, F), jnp.bfloat16),
        compiler_params=pltpu.CompilerParams(
            dimension_semantics=("arbitrary",),
        ),
    )(block_expert, block_valid, x_sorted, W1, b1r)
    fc2_spec = pltpu.PrefetchScalarGridSpec(
        num_scalar_prefetch=2,
        grid=(NB,),
        in_specs=[
            pl.BlockSpec((B, F), lambda i, be, va: (i, 0)),
            pl.BlockSpec((1, F, H), lambda i, be, va: (be[i], 0, 0)),
            pl.BlockSpec((1, 1, H), lambda i, be, va: (be[i], 0, 0)),
        ],
        out_specs=pl.BlockSpec((B, H), lambda i, be, va: (i, 0)),
    )
    return pl.pallas_call(
        _fc2_body,
        grid_spec=fc2_spec,
        out_shape=jax.ShapeDtypeStruct((P, H), jnp.float32),
        compiler_params=pltpu.CompilerParams(
            dimension_semantics=("arbitrary",),
        ),
    )(block_expert, block_valid, h_all, W2, b2r)


@jax.jit
def _moe(hidden_flat, probs, indices, W1, b1, W2, b2):
    eid = indices.astype(jnp.int32).reshape(A)
    p_flat = probs.reshape(A)
    b1r = b1.reshape(E, 1, F)
    b2r = b2.reshape(E, 1, H)
    hist = _sc_hist(eid)
    pos, be, valid, x_sorted = _sc_dispatch(eid, hist.reshape(NW * 16),
                                            hidden_flat)
    y_all = _grouped_mlp(x_sorted, be, valid, W1, b1r, W2, b2r)
    return _sc_combine(y_all, pos, p_flat)


def kernel(hidden_flat, probs, indices, W1, b1, W2, b2):
    return _moe(hidden_flat, probs, indices, W1, b1, W2, b2)
